# baseline (speedup 1.0000x reference)
import functools

import jax
import jax.numpy as jnp
from jax.experimental import pallas as pl
from jax.experimental.pallas import tpu as pltpu

_N_TOKENS = 32768
_D_MODEL = 2048
_NUM_EXPERTS = 64
_TOP_K = 8
_BT = 512


def _gate_body(x_ref, w_ref, out_w_ref, out_i_ref):
    x = x_ref[...]
    w = w_ref[...]
    logits = jax.lax.dot_general(
        x, w, (((1,), (1,)), ((), ())),
        preferred_element_type=jnp.float32,
    )
    iota = jax.lax.broadcasted_iota(jnp.int32, logits.shape, 1)
    vals = logits
    top_vals = []
    top_idxs = []
    for _ in range(_TOP_K):
        m = jnp.max(vals, axis=1, keepdims=True)
        idx = jnp.min(jnp.where(vals == m, iota, _NUM_EXPERTS), axis=1,
                      keepdims=True)
        top_vals.append(m)
        top_idxs.append(idx)
        vals = jnp.where(iota == idx, -jnp.inf, vals)
    tv = jnp.concatenate(top_vals, axis=1)
    ti = jnp.concatenate(top_idxs, axis=1)
    e = jnp.exp(tv - tv[:, 0:1])
    out_w_ref[...] = e / jnp.sum(e, axis=1, keepdims=True)
    out_i_ref[...] = ti


@jax.jit
def kernel(x, W):
    grid = (_N_TOKENS // _BT,)
    return pl.pallas_call(
        _gate_body,
        grid=grid,
        in_specs=[
            pl.BlockSpec((_BT, _D_MODEL), lambda i: (i, 0)),
            pl.BlockSpec((_NUM_EXPERTS, _D_MODEL), lambda i: (0, 0)),
        ],
        out_specs=[
            pl.BlockSpec((_BT, _TOP_K), lambda i: (i, 0)),
            pl.BlockSpec((_BT, _TOP_K), lambda i: (i, 0)),
        ],
        out_shape=[
            jax.ShapeDtypeStruct((_N_TOKENS, _TOP_K), jnp.float32),
            jax.ShapeDtypeStruct((_N_TOKENS, _TOP_K), jnp.int32),
        ],
    )(x, W)


# speedup vs baseline: 1.7132x; 1.7132x over previous
import jax
import jax.numpy as jnp
from jax.experimental import pallas as pl

_N_TOKENS = 32768
_D_MODEL = 2048
_NUM_EXPERTS = 64
_TOP_K = 8
_BT = 512


def _gate_body(x_ref, w_ref, out_w_ref, out_i_ref):
    x = x_ref[...]
    w = w_ref[...]
    vals = jax.lax.dot_general(
        w, x, (((1,), (1,)), ((), ())),
        preferred_element_type=jnp.float32,
    )
    iota = jax.lax.broadcasted_iota(jnp.int32, vals.shape, 0)
    top_vals = []
    top_idxs = []
    for _ in range(_TOP_K):
        m = jnp.max(vals, axis=0, keepdims=True)
        idx = jnp.min(jnp.where(vals == m, iota, _NUM_EXPERTS), axis=0,
                      keepdims=True)
        top_vals.append(m)
        top_idxs.append(idx)
        vals = jnp.where(iota == idx, -jnp.inf, vals)
    tv = jnp.concatenate(top_vals, axis=0)
    ti = jnp.concatenate(top_idxs, axis=0)
    e = jnp.exp(tv - tv[0:1])
    wgt = e / jnp.sum(e, axis=0, keepdims=True)
    eye = jnp.eye(_TOP_K, dtype=jnp.float32)
    out_w_ref[...] = jax.lax.dot_general(
        wgt, eye, (((0,), (0,)), ((), ())),
        preferred_element_type=jnp.float32)
    ti_f = ti.astype(jnp.float32)
    out_i_ref[...] = jax.lax.dot_general(
        ti_f, eye, (((0,), (0,)), ((), ())),
        preferred_element_type=jnp.float32).astype(jnp.int32)


@jax.jit
def kernel(x, W):
    grid = (_N_TOKENS // _BT,)
    return pl.pallas_call(
        _gate_body,
        grid=grid,
        in_specs=[
            pl.BlockSpec((_BT, _D_MODEL), lambda i: (i, 0)),
            pl.BlockSpec((_NUM_EXPERTS, _D_MODEL), lambda i: (0, 0)),
        ],
        out_specs=[
            pl.BlockSpec((_BT, _TOP_K), lambda i: (i, 0)),
            pl.BlockSpec((_BT, _TOP_K), lambda i: (i, 0)),
        ],
        out_shape=[
            jax.ShapeDtypeStruct((_N_TOKENS, _TOP_K), jnp.float32),
            jax.ShapeDtypeStruct((_N_TOKENS, _TOP_K), jnp.int32),
        ],
    )(x, W)


# speedup vs baseline: 2.0672x; 1.2066x over previous
import jax
import jax.numpy as jnp
from jax.experimental import pallas as pl

_N_TOKENS = 32768
_D_MODEL = 2048
_NUM_EXPERTS = 64
_TOP_K = 8
_BT = 1024


def _gate_body(x_ref, w_ref, out_w_ref, out_i_ref):
    x = x_ref[...]
    w = w_ref[...]
    vals = jax.lax.dot_general(
        w, x, (((1,), (1,)), ((), ())),
        preferred_element_type=jnp.float32,
    )
    iota = jax.lax.broadcasted_iota(jnp.int32, vals.shape, 0)
    top_vals = []
    top_idxs = []
    for _ in range(_TOP_K):
        m = jnp.max(vals, axis=0, keepdims=True)
        idx = jnp.min(jnp.where(vals == m, iota, _NUM_EXPERTS), axis=0,
                      keepdims=True)
        top_vals.append(m)
        top_idxs.append(idx)
        vals = jnp.where(iota == idx, -jnp.inf, vals)
    tv = jnp.concatenate(top_vals, axis=0)
    ti = jnp.concatenate(top_idxs, axis=0)
    e = jnp.exp(tv - tv[0:1])
    wgt = e / jnp.sum(e, axis=0, keepdims=True)
    eye = jnp.eye(_TOP_K, dtype=jnp.float32)
    out_w_ref[...] = jax.lax.dot_general(
        wgt, eye, (((0,), (0,)), ((), ())),
        preferred_element_type=jnp.float32)
    ti_f = ti.astype(jnp.float32)
    out_i_ref[...] = jax.lax.dot_general(
        ti_f, eye, (((0,), (0,)), ((), ())),
        preferred_element_type=jnp.float32).astype(jnp.int32)


@jax.jit
def kernel(x, W):
    grid = (_N_TOKENS // _BT,)
    return pl.pallas_call(
        _gate_body,
        grid=grid,
        in_specs=[
            pl.BlockSpec((_BT, _D_MODEL), lambda i: (i, 0)),
            pl.BlockSpec((_NUM_EXPERTS, _D_MODEL), lambda i: (0, 0)),
        ],
        out_specs=[
            pl.BlockSpec((_BT, _TOP_K), lambda i: (i, 0)),
            pl.BlockSpec((_BT, _TOP_K), lambda i: (i, 0)),
        ],
        out_shape=[
            jax.ShapeDtypeStruct((_N_TOKENS, _TOP_K), jnp.float32),
            jax.ShapeDtypeStruct((_N_TOKENS, _TOP_K), jnp.int32),
        ],
    )(x, W)


# speedup vs baseline: 2.2435x; 1.0853x over previous
import jax
import jax.numpy as jnp
from jax.experimental import pallas as pl

_N_TOKENS = 32768
_D_MODEL = 2048
_NUM_EXPERTS = 64
_TOP_K = 8
_BT = 2048


def _gate_body(x_ref, w_ref, out_w_ref, out_i_ref):
    x = x_ref[...]
    w = w_ref[...]
    vals = jax.lax.dot_general(
        w, x, (((1,), (1,)), ((), ())),
        preferred_element_type=jnp.float32,
    )
    iota = jax.lax.broadcasted_iota(jnp.int32, vals.shape, 0)
    top_vals = []
    top_idxs = []
    for _ in range(_TOP_K):
        m = jnp.max(vals, axis=0, keepdims=True)
        idx = jnp.min(jnp.where(vals == m, iota, _NUM_EXPERTS), axis=0,
                      keepdims=True)
        top_vals.append(m)
        top_idxs.append(idx)
        vals = jnp.where(iota == idx, -jnp.inf, vals)
    tv = jnp.concatenate(top_vals, axis=0)
    ti = jnp.concatenate(top_idxs, axis=0)
    e = jnp.exp(tv - tv[0:1])
    wgt = e / jnp.sum(e, axis=0, keepdims=True)
    eye = jnp.eye(_TOP_K, dtype=jnp.float32)
    out_w_ref[...] = jax.lax.dot_general(
        wgt, eye, (((0,), (0,)), ((), ())),
        preferred_element_type=jnp.float32)
    ti_f = ti.astype(jnp.float32)
    out_i_ref[...] = jax.lax.dot_general(
        ti_f, eye, (((0,), (0,)), ((), ())),
        preferred_element_type=jnp.float32).astype(jnp.int32)


@jax.jit
def kernel(x, W):
    grid = (_N_TOKENS // _BT,)
    return pl.pallas_call(
        _gate_body,
        grid=grid,
        in_specs=[
            pl.BlockSpec((_BT, _D_MODEL), lambda i: (i, 0)),
            pl.BlockSpec((_NUM_EXPERTS, _D_MODEL), lambda i: (0, 0)),
        ],
        out_specs=[
            pl.BlockSpec((_BT, _TOP_K), lambda i: (i, 0)),
            pl.BlockSpec((_BT, _TOP_K), lambda i: (i, 0)),
        ],
        out_shape=[
            jax.ShapeDtypeStruct((_N_TOKENS, _TOP_K), jnp.float32),
            jax.ShapeDtypeStruct((_N_TOKENS, _TOP_K), jnp.int32),
        ],
    )(x, W)
